# baseline (speedup 1.0000x reference)
import functools

import jax
import jax.numpy as jnp
from jax import lax
from jax.experimental import pallas as pl
from jax.experimental.pallas import tpu as pltpu
from jax.experimental.pallas import tpu_sc as plsc

TOP_K = 8
N_EXPERTS = 64
N_GROUP = 8
GROUP_SIZE = N_EXPERTS // N_GROUP
TOPK_GROUP = 4
SCALE = 2.5
H = 768
CH_TC = 1024
SUB = 128
NW = 32
SPLITS = (4096, 8192, 8192, 8192, 4096)


def _tc_scores_body(x_ref, w_ref, b_ref, out_ref):
    xb = x_ref[...]
    w = w_ref[...]
    logits_t = lax.dot_general(
        w, xb, (((1,), (1,)), ((), ())),
        preferred_element_type=jnp.float32,
    )
    scores = 1.0 / (1.0 + jnp.exp(-logits_t)) + b_ref[...]
    for k in range(CH_TC // SUB):
        out_ref[k] = scores[:, k * SUB:(k + 1) * SUB]


def _tc_scores(x, weight, bias_col, nh, chunk0):
    nch = nh // CH_TC
    return pl.pallas_call(
        _tc_scores_body,
        grid=(nch,),
        in_specs=[
            pl.BlockSpec((CH_TC, H), lambda i, c0=chunk0: (c0 + i, 0)),
            pl.BlockSpec((N_EXPERTS, H), lambda i: (0, 0)),
            pl.BlockSpec((N_EXPERTS, 1), lambda i: (0, 0)),
        ],
        out_specs=pl.BlockSpec((CH_TC // SUB, N_EXPERTS, SUB), lambda i: (i, 0, 0)),
        out_shape=jax.ShapeDtypeStruct((nh // SUB, N_EXPERTS, SUB), jnp.float32),
    )(x, weight, bias_col)


def _full_f(v):
    return jnp.full((16,), v, jnp.float32)


def _full_i(v):
    return jnp.full((16,), v, jnp.int32)


def _make_sc_body(ch):
    nsub = ch // SUB

    def _sc_route_body(st_hbm, bias_hbm, oidx_hbm, ow_hbm,
                       s_v, bias_v, comp_v, cidx_v, tw_v, obi_v, obw_v,
                       sem1, sem2):
        nc = 2
        wid = lax.axis_index("s") * nc + lax.axis_index("c")
        lane = lax.iota(jnp.int32, 16)
        neg = _full_f(-jnp.inf)

        c1 = pltpu.async_copy(bias_hbm, bias_v, sem1)
        c2 = pltpu.async_copy(st_hbm.at[pl.ds(wid * nsub, nsub)], s_v, sem2)
        c1.wait()
        c2.wait()

        def block(b, _):
            sub = b // (SUB // 16)
            lo = (b % (SUB // 16)) * 16
            tloc = b * 16 + lane
            subv = _full_i(0) + sub
            lov = _full_i(0) + lo + lane

            gs = []
            for g in range(N_GROUP):
                m1 = s_v[sub, g * GROUP_SIZE, pl.ds(lo, 16)]
                m2 = neg
                for e in range(1, GROUP_SIZE):
                    v = s_v[sub, g * GROUP_SIZE + e, pl.ds(lo, 16)]
                    m2 = jnp.maximum(m2, jnp.minimum(m1, v))
                    m1 = jnp.maximum(m1, v)
                gs.append(m1 + m2)

            gsel = []
            for _r in range(TOPK_GROUP):
                pairs = [(gs[g], _full_i(g)) for g in range(N_GROUP)]
                while len(pairs) > 1:
                    nxt = []
                    for a in range(0, len(pairs), 2):
                        (va, ia), (vb, ib) = pairs[a], pairs[a + 1]
                        c = vb > va
                        nxt.append((jnp.where(c, vb, va), jnp.where(c, ib, ia)))
                    pairs = nxt
                m, gid = pairs[0]
                for g in range(N_GROUP):
                    hit = jnp.logical_and(gs[g] == m, gid == _full_i(g))
                    gs[g] = jnp.where(hit, neg, gs[g])
                gsel.append(gid)
            for i, j in ((0, 1), (2, 3), (0, 2), (1, 3), (1, 2)):
                lo_ = jnp.minimum(gsel[i], gsel[j])
                hi_ = jnp.maximum(gsel[i], gsel[j])
                gsel[i], gsel[j] = lo_, hi_

            for a in range(TOPK_GROUP):
                base = gsel[a] * GROUP_SIZE
                for e in range(GROUP_SIZE):
                    eidx = base + e
                    val = plsc.load_gather(s_v, [subv, eidx, lov])
                    comp_v[a * GROUP_SIZE + e] = val
                    cidx_v[a * GROUP_SIZE + e] = eidx

            denom = _full_f(0.0)
            for k in range(TOP_K):
                pairs = []
                for r in range(0, TOPK_GROUP * GROUP_SIZE, 2):
                    va, vb = comp_v[r], comp_v[r + 1]
                    c = vb > va
                    pairs.append((jnp.where(c, vb, va),
                                  jnp.where(c, _full_i(r + 1), _full_i(r))))
                while len(pairs) > 1:
                    nxt = []
                    for a in range(0, len(pairs), 2):
                        (va, ia), (vb, ib) = pairs[a], pairs[a + 1]
                        c = vb > va
                        nxt.append((jnp.where(c, vb, va), jnp.where(c, ib, ia)))
                    pairs = nxt
                m, mi = pairs[0]
                evec = plsc.load_gather(cidx_v, [mi, lane])
                wvec = m - plsc.load_gather(bias_v, [evec])
                denom = denom + wvec
                tw_v[k] = wvec
                plsc.store_scatter(obi_v, [tloc, _full_i(k)], evec)
                plsc.store_scatter(comp_v, [mi, lane], neg)

            nrm = _full_f(SCALE) / (denom + _full_f(1e-20))
            for k in range(TOP_K):
                plsc.store_scatter(obw_v, [tloc, _full_i(k)], tw_v[k] * nrm)
            return 0

        lax.fori_loop(0, ch // 16, block, 0)

        pltpu.sync_copy(obi_v, oidx_hbm.at[pl.ds(wid * ch, ch)])
        pltpu.sync_copy(obw_v, ow_hbm.at[pl.ds(wid * ch, ch)])

    return _sc_route_body


def _sc_route(st, bias, nh):
    ch = nh // NW
    mesh = plsc.VectorSubcoreMesh(core_axis_name="c", subcore_axis_name="s")
    fn = pl.kernel(
        _make_sc_body(ch),
        out_type=[
            jax.ShapeDtypeStruct((nh, TOP_K), jnp.int32),
            jax.ShapeDtypeStruct((nh, TOP_K), jnp.float32),
        ],
        mesh=mesh,
        compiler_params=pltpu.CompilerParams(
            use_tc_tiling_on_sc=False, needs_layout_passes=False
        ),
        scratch_types=[
            pltpu.VMEM((ch // SUB, N_EXPERTS, SUB), jnp.float32),
            pltpu.VMEM((N_EXPERTS,), jnp.float32),
            pltpu.VMEM((32, 16), jnp.float32),
            pltpu.VMEM((32, 16), jnp.int32),
            pltpu.VMEM((8, 16), jnp.float32),
            pltpu.VMEM((ch, TOP_K), jnp.int32),
            pltpu.VMEM((ch, TOP_K), jnp.float32),
            pltpu.SemaphoreType.DMA,
            pltpu.SemaphoreType.DMA,
        ],
    )
    idx, wts = fn(st, bias)
    return idx, wts


def kernel(hidden_states, weight, e_score_correction_bias):
    bsz, seq_len, h = hidden_states.shape
    n = bsz * seq_len
    assert sum(SPLITS) == n
    x = hidden_states.reshape(n, h).astype(jnp.float32)
    w32 = weight.astype(jnp.float32)
    bias = e_score_correction_bias.astype(jnp.float32)
    bias_col = bias.reshape(N_EXPERTS, 1)
    idx_parts, w_parts = [], []
    chunk0 = 0
    for nh in SPLITS:
        st = _tc_scores(x, w32, bias_col, nh, chunk0)
        idx_h, w_h = _sc_route(st, bias, nh)
        idx_parts.append(idx_h)
        w_parts.append(w_h)
        chunk0 += nh // CH_TC
    return (jnp.concatenate(idx_parts, axis=0),
            jnp.concatenate(w_parts, axis=0))


# speedup vs baseline: 1.1561x; 1.1561x over previous
import functools

import jax
import jax.numpy as jnp
from jax import lax
from jax.experimental import pallas as pl
from jax.experimental.pallas import tpu as pltpu
from jax.experimental.pallas import tpu_sc as plsc

TOP_K = 8
N_EXPERTS = 64
N_GROUP = 8
GROUP_SIZE = N_EXPERTS // N_GROUP
TOPK_GROUP = 4
SCALE = 2.5
H = 768
CH_TC = 1024
SUB = 128
NW = 32
SPLITS = (8192, 8192, 8192, 8192)


def _tc_scores_body(x_ref, w_ref, b_ref, out_ref):
    xb = x_ref[...]
    w = w_ref[...]
    logits_t = lax.dot_general(
        w, xb, (((1,), (1,)), ((), ())),
        preferred_element_type=jnp.float32,
    )
    scores = 1.0 / (1.0 + jnp.exp(-logits_t)) + b_ref[...]
    for k in range(CH_TC // SUB):
        out_ref[k] = scores[:, k * SUB:(k + 1) * SUB]


def _tc_scores(x, weight, bias_col, nh, chunk0):
    nch = nh // CH_TC
    return pl.pallas_call(
        _tc_scores_body,
        grid=(nch,),
        in_specs=[
            pl.BlockSpec((CH_TC, H), lambda i, c0=chunk0: (c0 + i, 0)),
            pl.BlockSpec((N_EXPERTS, H), lambda i: (0, 0)),
            pl.BlockSpec((N_EXPERTS, 1), lambda i: (0, 0)),
        ],
        out_specs=pl.BlockSpec((CH_TC // SUB, N_EXPERTS, SUB), lambda i: (i, 0, 0)),
        out_shape=jax.ShapeDtypeStruct((nh // SUB, N_EXPERTS, SUB), jnp.float32),
    )(x, weight, bias_col)


def _full_f(v):
    return jnp.full((16,), v, jnp.float32)


def _full_i(v):
    return jnp.full((16,), v, jnp.int32)


def _make_sc_body(ch):
    nsub = ch // SUB

    def _sc_route_body(st_hbm, bias_hbm, oidx_hbm, ow_hbm,
                       s_v, bias_v, comp_v, cidx_v, tw_v, obi_v, obw_v,
                       sem1, sem2):
        nc = 2
        wid = lax.axis_index("s") * nc + lax.axis_index("c")
        lane = lax.iota(jnp.int32, 16)
        neg = _full_f(-jnp.inf)

        c1 = pltpu.async_copy(bias_hbm, bias_v, sem1)
        c2 = pltpu.async_copy(st_hbm.at[pl.ds(wid * nsub, nsub)], s_v, sem2)
        c1.wait()
        c2.wait()

        def block(b, _):
            sub = b // (SUB // 16)
            lo = (b % (SUB // 16)) * 16
            tloc = b * 16 + lane
            subv = _full_i(0) + sub
            lov = _full_i(0) + lo + lane

            gs = []
            for g in range(N_GROUP):
                m1 = s_v[sub, g * GROUP_SIZE, pl.ds(lo, 16)]
                m2 = neg
                for e in range(1, GROUP_SIZE):
                    v = s_v[sub, g * GROUP_SIZE + e, pl.ds(lo, 16)]
                    m2 = jnp.maximum(m2, jnp.minimum(m1, v))
                    m1 = jnp.maximum(m1, v)
                gs.append(m1 + m2)

            gsel = []
            for _r in range(TOPK_GROUP):
                pairs = [(gs[g], _full_i(g)) for g in range(N_GROUP)]
                while len(pairs) > 1:
                    nxt = []
                    for a in range(0, len(pairs), 2):
                        (va, ia), (vb, ib) = pairs[a], pairs[a + 1]
                        c = vb > va
                        nxt.append((jnp.where(c, vb, va), jnp.where(c, ib, ia)))
                    pairs = nxt
                m, gid = pairs[0]
                for g in range(N_GROUP):
                    hit = jnp.logical_and(gs[g] == m, gid == _full_i(g))
                    gs[g] = jnp.where(hit, neg, gs[g])
                gsel.append(gid)
            for i, j in ((0, 1), (2, 3), (0, 2), (1, 3), (1, 2)):
                lo_ = jnp.minimum(gsel[i], gsel[j])
                hi_ = jnp.maximum(gsel[i], gsel[j])
                gsel[i], gsel[j] = lo_, hi_

            for a in range(TOPK_GROUP):
                base = gsel[a] * GROUP_SIZE
                for e in range(GROUP_SIZE):
                    eidx = base + e
                    val = plsc.load_gather(s_v, [subv, eidx, lov])
                    comp_v[a * GROUP_SIZE + e] = val
                    cidx_v[a * GROUP_SIZE + e] = eidx

            denom = _full_f(0.0)
            for k in range(TOP_K):
                pairs = []
                for r in range(0, TOPK_GROUP * GROUP_SIZE, 2):
                    va, vb = comp_v[r], comp_v[r + 1]
                    c = vb > va
                    pairs.append((jnp.where(c, vb, va),
                                  jnp.where(c, _full_i(r + 1), _full_i(r))))
                while len(pairs) > 1:
                    nxt = []
                    for a in range(0, len(pairs), 2):
                        (va, ia), (vb, ib) = pairs[a], pairs[a + 1]
                        c = vb > va
                        nxt.append((jnp.where(c, vb, va), jnp.where(c, ib, ia)))
                    pairs = nxt
                m, mi = pairs[0]
                evec = plsc.load_gather(cidx_v, [mi, lane])
                wvec = m - plsc.load_gather(bias_v, [evec])
                denom = denom + wvec
                tw_v[k] = wvec
                plsc.store_scatter(obi_v, [tloc, _full_i(k)], evec)
                plsc.store_scatter(comp_v, [mi, lane], neg)

            nrm = _full_f(SCALE) / (denom + _full_f(1e-20))
            for k in range(TOP_K):
                plsc.store_scatter(obw_v, [tloc, _full_i(k)], tw_v[k] * nrm)
            return 0

        lax.fori_loop(0, ch // 16, block, 0)

        pltpu.sync_copy(obi_v, oidx_hbm.at[pl.ds(wid * ch, ch)])
        pltpu.sync_copy(obw_v, ow_hbm.at[pl.ds(wid * ch, ch)])

    return _sc_route_body


def _sc_route(st, bias, nh):
    ch = nh // NW
    mesh = plsc.VectorSubcoreMesh(core_axis_name="c", subcore_axis_name="s")
    fn = pl.kernel(
        _make_sc_body(ch),
        out_type=[
            jax.ShapeDtypeStruct((nh, TOP_K), jnp.int32),
            jax.ShapeDtypeStruct((nh, TOP_K), jnp.float32),
        ],
        mesh=mesh,
        compiler_params=pltpu.CompilerParams(
            use_tc_tiling_on_sc=False, needs_layout_passes=False
        ),
        scratch_types=[
            pltpu.VMEM((ch // SUB, N_EXPERTS, SUB), jnp.float32),
            pltpu.VMEM((N_EXPERTS,), jnp.float32),
            pltpu.VMEM((32, 16), jnp.float32),
            pltpu.VMEM((32, 16), jnp.int32),
            pltpu.VMEM((8, 16), jnp.float32),
            pltpu.VMEM((ch, TOP_K), jnp.int32),
            pltpu.VMEM((ch, TOP_K), jnp.float32),
            pltpu.SemaphoreType.DMA,
            pltpu.SemaphoreType.DMA,
        ],
    )
    idx, wts = fn(st, bias)
    return idx, wts


def kernel(hidden_states, weight, e_score_correction_bias):
    bsz, seq_len, h = hidden_states.shape
    n = bsz * seq_len
    assert sum(SPLITS) == n
    x = hidden_states.reshape(n, h).astype(jnp.float32)
    w32 = weight.astype(jnp.float32)
    bias = e_score_correction_bias.astype(jnp.float32)
    bias_col = bias.reshape(N_EXPERTS, 1)
    idx_parts, w_parts = [], []
    chunk0 = 0
    for nh in SPLITS:
        st = _tc_scores(x, w32, bias_col, nh, chunk0)
        idx_h, w_h = _sc_route(st, bias, nh)
        idx_parts.append(idx_h)
        w_parts.append(w_h)
        chunk0 += nh // CH_TC
    return (jnp.concatenate(idx_parts, axis=0),
            jnp.concatenate(w_parts, axis=0))
